# baseline (speedup 1.0000x reference)
import functools

import jax
import jax.numpy as jnp
from jax import lax
from jax.experimental import pallas as pl
from jax.experimental.pallas import tpu as pltpu
from jax.experimental.pallas import tpu_sc as plsc

_NC = 2
_NS = 16
_NW = _NC * _NS

_LT = 128
_ST = 8
_BT_PER_CHUNK = 2


@functools.lru_cache(maxsize=None)
def _make_gather(V, D, B, H):
    bpw = B // _NW
    nbt = bpw // _LT
    cbt = _BT_PER_CHUNK
    crows = cbt * _LT
    nhalf = nbt // cbt
    nchunk = H * nhalf
    nd = D // _ST

    mesh = plsc.VectorSubcoreMesh(core_axis_name="c", subcore_axis_name="s")

    @functools.partial(
        pl.kernel,
        out_type=jax.ShapeDtypeStruct((H, nd, B // _LT, _ST, _LT), jnp.float32),
        mesh=mesh,
        compiler_params=pltpu.CompilerParams(
            use_tc_tiling_on_sc=False, needs_layout_passes=False),
        scratch_types=[
            pltpu.VMEM((H, bpw), jnp.int32),
            pltpu.VMEM((crows, D), jnp.float32),
            pltpu.VMEM((crows, D), jnp.float32),
            pltpu.VMEM((nd, cbt, _ST, _LT), jnp.float32),
            pltpu.VMEM((nd, cbt, _ST, _LT), jnp.float32),
            pltpu.SemaphoreType.DMA,
            pltpu.SemaphoreType.DMA,
            pltpu.SemaphoreType.DMA,
            pltpu.SemaphoreType.DMA,
        ],
    )
    def gather(table_hbm, idxt_hbm, out_hbm, idx_v, gbuf_a, gbuf_b,
               sbuf_a, sbuf_b, gsem_a, gsem_b, osem_a, osem_b):
        wid = lax.axis_index("s") * _NC + lax.axis_index("c")
        b_base = wid * bpw

        pltpu.sync_copy(idxt_hbm.at[:, pl.ds(b_base, bpw)], idx_v)

        def fire_g(c, gbuf, sem):
            h = c // nhalf
            half = c % nhalf
            for j in range(cbt):
                pltpu.async_copy(
                    table_hbm.at[idx_v.at[h, pl.ds(half * crows + j * _LT, _LT)]],
                    gbuf.at[pl.ds(j * _LT, _LT)],
                    sem,
                )

        def drain_g(gbuf, sem):
            for j in range(cbt):
                pltpu.make_async_copy(
                    table_hbm.at[idx_v.at[0, pl.ds(0, _LT)]],
                    gbuf.at[pl.ds(j * _LT, _LT)],
                    sem,
                ).wait()

        lanes = lax.broadcasted_iota(jnp.int32, (16,), 0)

        def shuffle(gbuf, sbuf):
            def body(dt, carry):
                for t in range(cbt):
                    for ds in range(_ST):
                        col = dt * _ST + ds
                        cols = jnp.full((16,), col, jnp.int32)
                        for g in range(_LT // 16):
                            rows = lanes + (t * _LT + g * 16)
                            val = plsc.load_gather(gbuf, [rows, cols])
                            sbuf[dt, t, ds, pl.ds(g * 16, 16)] = val
                return carry
            lax.fori_loop(0, nd, body, 0)

        def dst(c):
            h = c // nhalf
            half = c % nhalf
            bt0 = wid * nbt + half * cbt
            return out_hbm.at[h, :, pl.ds(bt0, cbt)]

        def fire_s(c, sbuf, sem):
            pltpu.async_copy(sbuf, dst(c), sem)

        def drain_s(c, sbuf, sem):
            pltpu.make_async_copy(sbuf, dst(c), sem).wait()

        fire_g(0, gbuf_a, gsem_a)
        fire_g(1, gbuf_b, gsem_b)

        def half_step(k, i, gbuf, sbuf, gsem, osem):
            @pl.when(k > 0)
            def _():
                drain_s(i - 2, sbuf, osem)

            drain_g(gbuf, gsem)
            shuffle(gbuf, sbuf)

            @pl.when(i < nchunk - 2)
            def _():
                fire_g(i + 2, gbuf, gsem)

            fire_s(i, sbuf, osem)

        def body(k, carry):
            i = 2 * k
            half_step(k, i, gbuf_a, sbuf_a, gsem_a, osem_a)
            half_step(k, i + 1, gbuf_b, sbuf_b, gsem_b, osem_b)
            return carry

        lax.fori_loop(0, nchunk // 2, body, 0)
        drain_s(nchunk - 2, sbuf_a, osem_a)
        drain_s(nchunk - 1, sbuf_b, osem_b)

    return gather


def _scale_body(emb_ref, idf_ref, out_ref):
    out_ref[...] = emb_ref[...] * idf_ref[...]


def kernel(inputs, embeddings, idf):
    B, H = inputs.shape
    V, D = embeddings.shape

    scaled = pl.pallas_call(
        _scale_body,
        out_shape=jax.ShapeDtypeStruct((V, D), jnp.float32),
    )(embeddings, idf)

    idx_t = inputs.T
    out5 = _make_gather(V, D, B, H)(scaled, idx_t)
    return out5.transpose(2, 4, 0, 1, 3).reshape(B, H, D)


# speedup vs baseline: 1.8424x; 1.8424x over previous
import functools

import jax
import jax.numpy as jnp
from jax import lax
from jax.experimental import pallas as pl
from jax.experimental.pallas import tpu as pltpu
from jax.experimental.pallas import tpu_sc as plsc

_NC = 2
_NS = 16
_NW = _NC * _NS

_LT = 128
_ST = 8
_BT_PER_CHUNK = 2
_DW = 65


@functools.lru_cache(maxsize=None)
def _make_gather(V, D, B, H):
    bpw = B // _NW
    nbt = bpw // _LT
    cbt = _BT_PER_CHUNK
    crows = cbt * _LT
    nhalf = nbt // cbt
    nchunk = H * nhalf
    nd = D // _ST

    mesh = plsc.VectorSubcoreMesh(core_axis_name="c", subcore_axis_name="s")

    @functools.partial(
        pl.kernel,
        out_type=jax.ShapeDtypeStruct((H, nd, B // _LT, _ST, _LT), jnp.float32),
        mesh=mesh,
        compiler_params=pltpu.CompilerParams(
            use_tc_tiling_on_sc=False, needs_layout_passes=False),
        scratch_types=[
            pltpu.VMEM((H, bpw), jnp.int32),
            pltpu.VMEM((crows, _DW), jnp.float32),
            pltpu.VMEM((crows, _DW), jnp.float32),
            pltpu.VMEM((nd, cbt, _ST, _LT), jnp.float32),
            pltpu.VMEM((nd, cbt, _ST, _LT), jnp.float32),
            pltpu.SemaphoreType.DMA,
            pltpu.SemaphoreType.DMA,
            pltpu.SemaphoreType.DMA,
            pltpu.SemaphoreType.DMA,
        ],
    )
    def gather(table_hbm, idxt_hbm, out_hbm, idx_v, gbuf_a, gbuf_b,
               sbuf_a, sbuf_b, gsem_a, gsem_b, osem_a, osem_b):
        wid = lax.axis_index("s") * _NC + lax.axis_index("c")
        b_base = wid * bpw

        pltpu.sync_copy(idxt_hbm.at[:, pl.ds(b_base, bpw)], idx_v)

        def fire_g(c, gbuf, sem):
            h = c // nhalf
            half = c % nhalf
            for j in range(cbt):
                pltpu.async_copy(
                    table_hbm.at[idx_v.at[h, pl.ds(half * crows + j * _LT, _LT)]],
                    gbuf.at[pl.ds(j * _LT, _LT)],
                    sem,
                )

        def drain_g(gbuf, sem):
            for j in range(cbt):
                pltpu.make_async_copy(
                    table_hbm.at[idx_v.at[0, pl.ds(0, _LT)]],
                    gbuf.at[pl.ds(j * _LT, _LT)],
                    sem,
                ).wait()

        lanes = lax.broadcasted_iota(jnp.int32, (16,), 0)

        def shuffle(gbuf, sbuf):
            def body(dt, carry):
                for t in range(cbt):
                    for ds in range(_ST):
                        col = dt * _ST + ds
                        cols = jnp.full((16,), col, jnp.int32)
                        for g in range(_LT // 16):
                            rows = lanes + (t * _LT + g * 16)
                            val = plsc.load_gather(gbuf, [rows, cols])
                            sbuf[dt, t, ds, pl.ds(g * 16, 16)] = val
                return carry
            lax.fori_loop(0, nd, body, 0)

        def dst(c):
            h = c // nhalf
            half = c % nhalf
            bt0 = wid * nbt + half * cbt
            return out_hbm.at[h, :, pl.ds(bt0, cbt)]

        def fire_s(c, sbuf, sem):
            pltpu.async_copy(sbuf, dst(c), sem)

        def drain_s(c, sbuf, sem):
            pltpu.make_async_copy(sbuf, dst(c), sem).wait()

        fire_g(0, gbuf_a, gsem_a)
        fire_g(1, gbuf_b, gsem_b)

        def half_step(k, i, gbuf, sbuf, gsem, osem):
            @pl.when(k > 0)
            def _():
                drain_s(i - 2, sbuf, osem)

            drain_g(gbuf, gsem)
            shuffle(gbuf, sbuf)

            @pl.when(i < nchunk - 2)
            def _():
                fire_g(i + 2, gbuf, gsem)

            fire_s(i, sbuf, osem)

        def body(k, carry):
            i = 2 * k
            half_step(k, i, gbuf_a, sbuf_a, gsem_a, osem_a)
            half_step(k, i + 1, gbuf_b, sbuf_b, gsem_b, osem_b)
            return carry

        lax.fori_loop(0, nchunk // 2, body, 0)
        drain_s(nchunk - 2, sbuf_a, osem_a)
        drain_s(nchunk - 1, sbuf_b, osem_b)

    return gather


def _scale_pad_body(emb_ref, idf_ref, out_ref):
    scaled = emb_ref[...] * idf_ref[...]
    pad = jnp.zeros((scaled.shape[0], _DW - scaled.shape[1]), scaled.dtype)
    out_ref[...] = jnp.concatenate([scaled, pad], axis=-1)


def kernel(inputs, embeddings, idf):
    B, H = inputs.shape
    V, D = embeddings.shape

    scaled = pl.pallas_call(
        _scale_pad_body,
        out_shape=jax.ShapeDtypeStruct((V, _DW), jnp.float32),
    )(embeddings, idf)

    idx_t = inputs.T
    out5 = _make_gather(V, D, B, H)(scaled, idx_t)
    return out5.transpose(2, 4, 0, 1, 3).reshape(B, H, D)


# speedup vs baseline: 3.1313x; 1.6996x over previous
import functools

import jax
import jax.numpy as jnp
from jax import lax
from jax.experimental import pallas as pl
from jax.experimental.pallas import tpu as pltpu
from jax.experimental.pallas import tpu_sc as plsc

_NC = 2
_NS = 16
_NW = _NC * _NS

_LT = 128
_ST = 8
_BT_PER_CHUNK = 2
_DW = 65


@functools.lru_cache(maxsize=None)
def _make_gather(V, D, B, H):
    bpw = B // _NW
    nbt = bpw // _LT
    cbt = _BT_PER_CHUNK
    crows = cbt * _LT
    nhalf = nbt // cbt
    nchunk = H * nhalf
    nd = D // _ST

    mesh = plsc.VectorSubcoreMesh(core_axis_name="c", subcore_axis_name="s")

    @functools.partial(
        pl.kernel,
        out_type=jax.ShapeDtypeStruct((H, nd, B // _LT, _ST, _LT), jnp.float32),
        mesh=mesh,
        compiler_params=pltpu.CompilerParams(
            use_tc_tiling_on_sc=False, needs_layout_passes=False),
        scratch_types=[
            pltpu.VMEM((H, bpw), jnp.int32),
            pltpu.VMEM((crows, D), jnp.float32),
            pltpu.VMEM((crows, D), jnp.float32),
            pltpu.VMEM((nd, cbt, _ST, _LT), jnp.float32),
            pltpu.VMEM((nd, cbt, _ST, _LT), jnp.float32),
            pltpu.SemaphoreType.DMA,
            pltpu.SemaphoreType.DMA,
            pltpu.SemaphoreType.DMA,
            pltpu.SemaphoreType.DMA,
        ],
    )
    def gather(table_hbm, idxt_hbm, out_hbm, idx_v, gbuf_a, gbuf_b,
               sbuf_a, sbuf_b, gsem_a, gsem_b, osem_a, osem_b):
        wid = lax.axis_index("s") * _NC + lax.axis_index("c")
        b_base = wid * bpw

        pltpu.sync_copy(idxt_hbm.at[:, pl.ds(b_base, bpw)], idx_v)

        def fire_g(c, gbuf, sem):
            h = c // nhalf
            half = c % nhalf
            for j in range(cbt):
                pltpu.async_copy(
                    table_hbm.at[idx_v.at[h, pl.ds(half * crows + j * _LT, _LT)]],
                    gbuf.at[pl.ds(j * _LT, _LT)],
                    sem,
                )

        def drain_g(gbuf, sem):
            for j in range(cbt):
                pltpu.make_async_copy(
                    table_hbm.at[idx_v.at[0, pl.ds(0, _LT)]],
                    gbuf.at[pl.ds(j * _LT, _LT)],
                    sem,
                ).wait()

        lanes = lax.broadcasted_iota(jnp.int32, (16,), 0)

        def shuffle(gbuf, sbuf):
            def body(s, carry):
                rot = (lanes + s) & 15
                for d0 in range(0, D, 16):
                    cols = rot + d0
                    dtv = cols >> 3
                    dsv = cols & 7
                    for t in range(cbt):
                        tv = jnp.full((16,), t, jnp.int32)
                        for g in range(_LT // 16):
                            rows = lanes + (t * _LT + g * 16)
                            bsv = lanes + (g * 16)
                            val = plsc.load_gather(gbuf, [rows, cols])
                            plsc.store_scatter(sbuf, [dtv, tv, dsv, bsv], val)
                return carry
            lax.fori_loop(0, 16, body, 0)

        def dst(c):
            h = c // nhalf
            half = c % nhalf
            bt0 = wid * nbt + half * cbt
            return out_hbm.at[h, :, pl.ds(bt0, cbt)]

        def fire_s(c, sbuf, sem):
            pltpu.async_copy(sbuf, dst(c), sem)

        def drain_s(c, sbuf, sem):
            pltpu.make_async_copy(sbuf, dst(c), sem).wait()

        fire_g(0, gbuf_a, gsem_a)
        fire_g(1, gbuf_b, gsem_b)

        def half_step(k, i, gbuf, sbuf, gsem, osem):
            @pl.when(k > 0)
            def _():
                drain_s(i - 2, sbuf, osem)

            drain_g(gbuf, gsem)
            shuffle(gbuf, sbuf)

            @pl.when(i < nchunk - 2)
            def _():
                fire_g(i + 2, gbuf, gsem)

            fire_s(i, sbuf, osem)

        def body(k, carry):
            i = 2 * k
            half_step(k, i, gbuf_a, sbuf_a, gsem_a, osem_a)
            half_step(k, i + 1, gbuf_b, sbuf_b, gsem_b, osem_b)
            return carry

        lax.fori_loop(0, nchunk // 2, body, 0)
        drain_s(nchunk - 2, sbuf_a, osem_a)
        drain_s(nchunk - 1, sbuf_b, osem_b)

    return gather


def _scale_body(emb_ref, idf_ref, out_ref):
    out_ref[...] = emb_ref[...] * idf_ref[...]


def kernel(inputs, embeddings, idf):
    B, H = inputs.shape
    V, D = embeddings.shape

    scaled = pl.pallas_call(
        _scale_body,
        out_shape=jax.ShapeDtypeStruct((V, D), jnp.float32),
    )(embeddings, idf)

    idx_t = inputs.T
    out5 = _make_gather(V, D, B, H)(scaled, idx_t)
    return out5.transpose(2, 4, 0, 1, 3).reshape(B, H, D)


# speedup vs baseline: 3.1416x; 1.0033x over previous
import functools

import jax
import jax.numpy as jnp
from jax import lax
from jax.experimental import pallas as pl
from jax.experimental.pallas import tpu as pltpu
from jax.experimental.pallas import tpu_sc as plsc

_NC = 2
_NS = 16
_NW = _NC * _NS

_LT = 128
_ST = 8
_BT_PER_CHUNK = 2
_DW = 65


@functools.lru_cache(maxsize=None)
def _make_gather(V, D, B, H):
    bpw = B // _NW
    nbt = bpw // _LT
    cbt = _BT_PER_CHUNK
    crows = cbt * _LT
    nhalf = nbt // cbt
    nchunk = H * nhalf
    nd = D // _ST

    mesh = plsc.VectorSubcoreMesh(core_axis_name="c", subcore_axis_name="s")

    @functools.partial(
        pl.kernel,
        out_type=jax.ShapeDtypeStruct((H, nd, B // _LT, _ST, _LT), jnp.float32),
        mesh=mesh,
        compiler_params=pltpu.CompilerParams(
            use_tc_tiling_on_sc=False, needs_layout_passes=False),
        scratch_types=[
            pltpu.VMEM((H, bpw), jnp.int32),
            pltpu.VMEM((crows, D), jnp.float32),
            pltpu.VMEM((crows, D), jnp.float32),
            pltpu.VMEM((nd, cbt, _ST, _LT), jnp.float32),
            pltpu.VMEM((nd, cbt, _ST, _LT), jnp.float32),
            pltpu.SemaphoreType.DMA,
            pltpu.SemaphoreType.DMA,
            pltpu.SemaphoreType.DMA,
            pltpu.SemaphoreType.DMA,
        ],
    )
    def gather(table_hbm, idxt_hbm, out_hbm, idx_v, gbuf_a, gbuf_b,
               sbuf_a, sbuf_b, gsem_a, gsem_b, osem_a, osem_b):
        wid = lax.axis_index("s") * _NC + lax.axis_index("c")
        b_base = wid * bpw

        pltpu.sync_copy(idxt_hbm.at[:, pl.ds(b_base, bpw)], idx_v)

        def fire_g(c, gbuf, sem):
            h = c // nhalf
            half = c % nhalf
            for j in range(cbt):
                pltpu.async_copy(
                    table_hbm.at[idx_v.at[h, pl.ds(half * crows + j * _LT, _LT)]],
                    gbuf.at[pl.ds(j * _LT, _LT)],
                    sem,
                )

        def drain_g(gbuf, sem):
            for j in range(cbt):
                pltpu.make_async_copy(
                    table_hbm.at[idx_v.at[0, pl.ds(0, _LT)]],
                    gbuf.at[pl.ds(j * _LT, _LT)],
                    sem,
                ).wait()

        lanes = lax.broadcasted_iota(jnp.int32, (16,), 0)
        rows_tab = [lanes + (t * _LT + g * 16)
                    for t in range(cbt) for g in range(_LT // 16)]
        tvs = [jnp.full((16,), t, jnp.int32) for t in range(cbt)]

        def shuffle(gbuf, sbuf):
            def body(s, carry):
                rot = (lanes + s) & 15
                for d0 in range(0, D, 16):
                    cols = rot + d0
                    dtv = cols >> 3
                    dsv = cols & 7
                    for t in range(cbt):
                        for g in range(_LT // 16):
                            val = plsc.load_gather(
                                gbuf, [rows_tab[t * (_LT // 16) + g], cols])
                            plsc.store_scatter(
                                sbuf, [dtv, tvs[t], dsv, rows_tab[g]], val)
                return carry
            lax.fori_loop(0, 16, body, 0)

        def dst(c):
            h = c // nhalf
            half = c % nhalf
            bt0 = wid * nbt + half * cbt
            return out_hbm.at[h, :, pl.ds(bt0, cbt)]

        def fire_s(c, sbuf, sem):
            pltpu.async_copy(sbuf, dst(c), sem)

        def drain_s(c, sbuf, sem):
            pltpu.make_async_copy(sbuf, dst(c), sem).wait()

        fire_g(0, gbuf_a, gsem_a)
        fire_g(1, gbuf_b, gsem_b)

        def half_step(k, i, gbuf, sbuf, gsem, osem):
            @pl.when(k > 0)
            def _():
                drain_s(i - 2, sbuf, osem)

            drain_g(gbuf, gsem)
            shuffle(gbuf, sbuf)

            @pl.when(i < nchunk - 2)
            def _():
                fire_g(i + 2, gbuf, gsem)

            fire_s(i, sbuf, osem)

        def body(k, carry):
            i = 2 * k
            half_step(k, i, gbuf_a, sbuf_a, gsem_a, osem_a)
            half_step(k, i + 1, gbuf_b, sbuf_b, gsem_b, osem_b)
            return carry

        lax.fori_loop(0, nchunk // 2, body, 0)
        drain_s(nchunk - 2, sbuf_a, osem_a)
        drain_s(nchunk - 1, sbuf_b, osem_b)

    return gather


def _scale_body(emb_ref, idf_ref, out_ref):
    out_ref[...] = emb_ref[...] * idf_ref[...]


def kernel(inputs, embeddings, idf):
    B, H = inputs.shape
    V, D = embeddings.shape

    scaled = pl.pallas_call(
        _scale_body,
        out_shape=jax.ShapeDtypeStruct((V, D), jnp.float32),
    )(embeddings, idf)

    idx_t = inputs.T
    out5 = _make_gather(V, D, B, H)(scaled, idx_t)
    return out5.transpose(2, 4, 0, 1, 3).reshape(B, H, D)


# speedup vs baseline: 3.6700x; 1.1682x over previous
import functools

import jax
import jax.numpy as jnp
from jax import lax
from jax.experimental import pallas as pl
from jax.experimental.pallas import tpu as pltpu
from jax.experimental.pallas import tpu_sc as plsc

_NC = 2
_NS = 16
_NW = _NC * _NS

_LT = 128
_ST = 8
_BT_PER_CHUNK = 2
_DW = 65


@functools.lru_cache(maxsize=None)
def _make_gather(V, D, B, H):
    bpw = B // _NW
    nbt = bpw // _LT
    cbt = _BT_PER_CHUNK
    crows = cbt * _LT
    nhalf = nbt // cbt
    nchunk = H * nhalf
    nd = D // _ST

    mesh = plsc.VectorSubcoreMesh(core_axis_name="c", subcore_axis_name="s")

    @functools.partial(
        pl.kernel,
        out_type=jax.ShapeDtypeStruct((H, nd, B // _LT, _ST, _LT), jnp.float32),
        mesh=mesh,
        compiler_params=pltpu.CompilerParams(
            use_tc_tiling_on_sc=False, needs_layout_passes=False),
        scratch_types=[
            pltpu.VMEM((H, bpw), jnp.int32),
            pltpu.VMEM((crows, D), jnp.float32),
            pltpu.VMEM((crows, D), jnp.float32),
            pltpu.VMEM((nd, cbt, _ST, _LT), jnp.float32),
            pltpu.VMEM((nd, cbt, _ST, _LT), jnp.float32),
            pltpu.SemaphoreType.DMA,
            pltpu.SemaphoreType.DMA,
            pltpu.SemaphoreType.DMA,
            pltpu.SemaphoreType.DMA,
        ],
    )
    def gather(table_hbm, idxt_hbm, out_hbm, idx_v, gbuf_a, gbuf_b,
               sbuf_a, sbuf_b, gsem_a, gsem_b, osem_a, osem_b):
        wid = lax.axis_index("s") * _NC + lax.axis_index("c")
        b_base = wid * bpw

        pltpu.sync_copy(idxt_hbm.at[:, pl.ds(b_base, bpw)], idx_v)

        def fire_g(c, gbuf, sem):
            h = c // nhalf
            half = c % nhalf
            for j in range(cbt):
                pltpu.async_copy(
                    table_hbm.at[idx_v.at[h, pl.ds(half * crows + j * _LT, _LT)]],
                    gbuf.at[pl.ds(j * _LT, _LT)],
                    sem,
                )

        def drain_g(gbuf, sem):
            for j in range(cbt):
                pltpu.make_async_copy(
                    table_hbm.at[idx_v.at[0, pl.ds(0, _LT)]],
                    gbuf.at[pl.ds(j * _LT, _LT)],
                    sem,
                ).wait()

        lanes = lax.broadcasted_iota(jnp.int32, (16,), 0)
        rows_tab = [lanes + (t * _LT + g * 16)
                    for t in range(cbt) for g in range(_LT // 16)]
        tvs = [jnp.full((16,), t, jnp.int32) for t in range(cbt)]

        def shuffle(gbuf, sbuf):
            def body(s, carry):
                rot = (lanes + s) & 15
                for d0 in range(0, D, 16):
                    cols = rot + d0
                    dtv = cols >> 3
                    dsv = cols & 7
                    for t in range(cbt):
                        vals = [
                            plsc.load_gather(
                                gbuf, [rows_tab[t * (_LT // 16) + g], cols])
                            for g in range(_LT // 16)
                        ]
                        for g in range(_LT // 16):
                            plsc.store_scatter(
                                sbuf, [dtv, tvs[t], dsv, rows_tab[g]], vals[g])
                return carry
            lax.fori_loop(0, 16, body, 0)

        def dst(c):
            h = c // nhalf
            half = c % nhalf
            bt0 = wid * nbt + half * cbt
            return out_hbm.at[h, :, pl.ds(bt0, cbt)]

        def fire_s(c, sbuf, sem):
            pltpu.async_copy(sbuf, dst(c), sem)

        def drain_s(c, sbuf, sem):
            pltpu.make_async_copy(sbuf, dst(c), sem).wait()

        fire_g(0, gbuf_a, gsem_a)
        fire_g(1, gbuf_b, gsem_b)

        def half_step(k, i, gbuf, sbuf, gsem, osem):
            @pl.when(k > 0)
            def _():
                drain_s(i - 2, sbuf, osem)

            drain_g(gbuf, gsem)
            shuffle(gbuf, sbuf)

            @pl.when(i < nchunk - 2)
            def _():
                fire_g(i + 2, gbuf, gsem)

            fire_s(i, sbuf, osem)

        def body(k, carry):
            i = 2 * k
            half_step(k, i, gbuf_a, sbuf_a, gsem_a, osem_a)
            half_step(k, i + 1, gbuf_b, sbuf_b, gsem_b, osem_b)
            return carry

        lax.fori_loop(0, nchunk // 2, body, 0)
        drain_s(nchunk - 2, sbuf_a, osem_a)
        drain_s(nchunk - 1, sbuf_b, osem_b)

    return gather


def _scale_body(emb_ref, idf_ref, out_ref):
    out_ref[...] = emb_ref[...] * idf_ref[...]


def kernel(inputs, embeddings, idf):
    B, H = inputs.shape
    V, D = embeddings.shape

    scaled = pl.pallas_call(
        _scale_body,
        out_shape=jax.ShapeDtypeStruct((V, D), jnp.float32),
    )(embeddings, idf)

    idx_t = inputs.T
    out5 = _make_gather(V, D, B, H)(scaled, idx_t)
    return out5.transpose(2, 4, 0, 1, 3).reshape(B, H, D)
